# baseline (speedup 1.0000x reference)
import functools

import jax
import jax.numpy as jnp
from jax import lax
from jax.experimental import pallas as pl
from jax.experimental.pallas import tpu as pltpu
from jax.experimental.pallas import tpu_sc as plsc

INTD = 32
HD = 96
RBLK = 512

_N_INT_BLK = 1
_N_TEST_BLK = 20
_N_Q_BLK = 196
_N_TAG_BLK = 2
_NBLK = _N_INT_BLK + _N_TEST_BLK + _N_Q_BLK + _N_TAG_BLK
_OFF_TEST = _N_INT_BLK * RBLK
_OFF_Q = _OFF_TEST + _N_TEST_BLK * RBLK
_OFF_TAG = _OFF_Q + _N_Q_BLK * RBLK
_PROWS = _NBLK * RBLK


def _proj_body(xi_ref, xt_ref, xq_ref, xg_ref, w_ref, b_ref, o_ref):
    g = pl.program_id(0)

    @pl.when(g == 0)
    def _():
        r = jnp.dot(xi_ref[...], w_ref[0], preferred_element_type=jnp.float32)
        o_ref[0:3, :] = r + b_ref[...]

    @pl.when(jnp.logical_and(g >= _N_INT_BLK, g < _N_INT_BLK + _N_TEST_BLK))
    def _():
        o_ref[...] = jnp.dot(xt_ref[...], w_ref[1], preferred_element_type=jnp.float32)

    @pl.when(jnp.logical_and(g >= _N_INT_BLK + _N_TEST_BLK, g < _NBLK - _N_TAG_BLK))
    def _():
        o_ref[...] = jnp.dot(xq_ref[...], w_ref[2], preferred_element_type=jnp.float32)

    @pl.when(g >= _NBLK - _N_TAG_BLK)
    def _():
        o_ref[...] = jnp.dot(xg_ref[...], w_ref[3], preferred_element_type=jnp.float32)


def _project(emb_int, emb_test, emb_q, emb_tag, w4, b2):
    return pl.pallas_call(
        _proj_body,
        grid=(_NBLK,),
        in_specs=[
            pl.BlockSpec((3, INTD), lambda g: (0, 0)),
            pl.BlockSpec((RBLK, INTD), lambda g: (jnp.clip(g - _N_INT_BLK, 0, _N_TEST_BLK - 1), 0)),
            pl.BlockSpec((RBLK, INTD), lambda g: (jnp.clip(g - _N_INT_BLK - _N_TEST_BLK, 0, _N_Q_BLK - 1), 0)),
            pl.BlockSpec((RBLK, INTD), lambda g: (jnp.clip(g - (_NBLK - _N_TAG_BLK), 0, _N_TAG_BLK - 1), 0)),
            pl.BlockSpec((4, INTD, HD), lambda g: (0, 0, 0)),
            pl.BlockSpec((1, HD), lambda g: (0, 0)),
        ],
        out_specs=pl.BlockSpec((RBLK, HD), lambda g: (g, 0)),
        out_shape=jax.ShapeDtypeStruct((_PROWS, HD), jnp.float32),
    )(emb_int, emb_test, emb_q, emb_tag, w4, b2)


@functools.lru_cache(maxsize=None)
def _make_gather_sum(ntok):
    info = plsc.get_sparse_core_info()
    nc, ns = info.num_cores, info.num_subcores
    nw = nc * ns
    tpw = ntok // nw
    C = 128
    nchunk = tpw // C
    mesh = plsc.VectorSubcoreMesh(core_axis_name="c", subcore_axis_name="s")

    @functools.partial(
        pl.kernel,
        mesh=mesh,
        compiler_params=pltpu.CompilerParams(use_tc_tiling_on_sc=False),
        out_type=jax.ShapeDtypeStruct((ntok, HD), jnp.float32),
        scratch_types=[
            pltpu.VMEM((C,), jnp.int32),
            pltpu.VMEM((C,), jnp.int32),
            pltpu.VMEM((C,), jnp.int32),
            pltpu.VMEM((C,), jnp.int32),
            pltpu.VMEM((C, HD), jnp.float32),
            pltpu.VMEM((C, HD), jnp.float32),
            pltpu.VMEM((C, HD), jnp.float32),
            pltpu.VMEM((C, HD), jnp.float32),
            pltpu.SemaphoreType.DMA,
        ],
    )
    def gather_sum(p_hbm, ii_hbm, it_hbm, iq_hbm, ig_hbm, out_hbm,
                   vii, vit, viq, vig, r0, r1, r2, r3, sem):
        wid = lax.axis_index("s") * nc + lax.axis_index("c")
        base = wid * tpw

        def chunk(ci, carry):
            off = base + ci * C
            pltpu.sync_copy(ii_hbm.at[pl.ds(off, C)], vii)
            pltpu.sync_copy(it_hbm.at[pl.ds(off, C)], vit)
            pltpu.sync_copy(iq_hbm.at[pl.ds(off, C)], viq)
            pltpu.sync_copy(ig_hbm.at[pl.ds(off, C)], vig)
            for j in range(C // 16):
                sl = pl.ds(j * 16, 16)
                vit[sl] = vit[sl] + _OFF_TEST
                viq[sl] = viq[sl] + _OFF_Q
                vig[sl] = vig[sl] + _OFF_TAG
            cp0 = pltpu.async_copy(p_hbm.at[vii], r0, sem)
            cp1 = pltpu.async_copy(p_hbm.at[vit], r1, sem)
            cp2 = pltpu.async_copy(p_hbm.at[viq], r2, sem)
            cp3 = pltpu.async_copy(p_hbm.at[vig], r3, sem)
            cp0.wait()
            cp1.wait()
            cp2.wait()
            cp3.wait()

            def add_one(rr, c2):
                for k in range(HD // 16):
                    sl = pl.ds(k * 16, 16)
                    r0[rr, sl] = r0[rr, sl] + r1[rr, sl] + r2[rr, sl] + r3[rr, sl]
                return c2

            lax.fori_loop(0, C, add_one, 0)
            pltpu.sync_copy(r0, out_hbm.at[pl.ds(off, C)])
            return carry

        lax.fori_loop(0, nchunk, chunk, 0)

    return gather_sum


def kernel(testId, assessmentItemID, KnowledgeTag, answerCode, mask, interaction,
           emb_interaction, emb_test, emb_question, emb_tag, W, b):
    bsz, seq = interaction.shape
    ntok = bsz * seq
    ii = interaction.reshape(-1).astype(jnp.int32)
    it = testId.reshape(-1).astype(jnp.int32)
    iq = assessmentItemID.reshape(-1).astype(jnp.int32)
    ig = KnowledgeTag.reshape(-1).astype(jnp.int32)

    P = _project(emb_interaction, emb_test, emb_question, emb_tag,
                 W.reshape(4, INTD, HD), b.reshape(1, HD))
    Xf = _make_gather_sum(ntok)(P, ii, it, iq, ig)
    return (Xf.reshape(bsz, seq, HD), bsz)


# speedup vs baseline: 8.1675x; 8.1675x over previous
import functools

import jax
import jax.numpy as jnp
from jax import lax
from jax.experimental import pallas as pl
from jax.experimental.pallas import tpu as pltpu
from jax.experimental.pallas import tpu_sc as plsc

INTD = 32
HD = 96
RBLK = 512

_N_TEST_BLK = 20
_N_Q_BLK = 196
_N_TAG_BLK = 2
_NBLK = _N_TEST_BLK + _N_Q_BLK + _N_TAG_BLK
_OFF_Q = _N_TEST_BLK * RBLK
_TQ_ROWS = (_N_TEST_BLK + _N_Q_BLK) * RBLK
_G_ROWS = _N_TAG_BLK * RBLK
_TI_ROWS = 3 * _G_ROWS


def _proj_body(xt_ref, xq_ref, xg_ref, w_ref, otq_ref, og_ref):
    g = pl.program_id(0)

    @pl.when(g < _N_TEST_BLK)
    def _():
        otq_ref[...] = jnp.dot(xt_ref[...], w_ref[1], preferred_element_type=jnp.float32)

    @pl.when(jnp.logical_and(g >= _N_TEST_BLK, g < _N_TEST_BLK + _N_Q_BLK))
    def _():
        otq_ref[...] = jnp.dot(xq_ref[...], w_ref[2], preferred_element_type=jnp.float32)

    @pl.when(g >= _N_TEST_BLK + _N_Q_BLK)
    def _():
        og_ref[...] = jnp.dot(xg_ref[...], w_ref[3], preferred_element_type=jnp.float32)


def _project(emb_test, emb_q, emb_tag, w4):
    return pl.pallas_call(
        _proj_body,
        grid=(_NBLK,),
        in_specs=[
            pl.BlockSpec((RBLK, INTD), lambda g: (jnp.clip(g, 0, _N_TEST_BLK - 1), 0)),
            pl.BlockSpec((RBLK, INTD), lambda g: (jnp.clip(g - _N_TEST_BLK, 0, _N_Q_BLK - 1), 0)),
            pl.BlockSpec((RBLK, INTD), lambda g: (jnp.clip(g - _N_TEST_BLK - _N_Q_BLK, 0, _N_TAG_BLK - 1), 0)),
            pl.BlockSpec((4, INTD, HD), lambda g: (0, 0, 0)),
        ],
        out_specs=[
            pl.BlockSpec((RBLK, HD), lambda g: (jnp.clip(g, 0, _N_TEST_BLK + _N_Q_BLK - 1), 0)),
            pl.BlockSpec((RBLK, HD), lambda g: (jnp.clip(g - _N_TEST_BLK - _N_Q_BLK, 0, _N_TAG_BLK - 1), 0)),
        ],
        out_shape=[
            jax.ShapeDtypeStruct((_TQ_ROWS, HD), jnp.float32),
            jax.ShapeDtypeStruct((_G_ROWS, HD), jnp.float32),
        ],
    )(emb_test, emb_q, emb_tag, w4)


def _combine_body(pg_ref, xi_ref, w_ref, b_ref, o_ref):
    pint = jnp.dot(xi_ref[...], w_ref[0], preferred_element_type=jnp.float32) + b_ref[...]
    pg = pg_ref[...]
    o_ref[...] = (pg[:, None, :] + pint[None, :, :]).reshape(3 * RBLK, HD)


def _combine(pg, emb_int, w4, b2):
    return pl.pallas_call(
        _combine_body,
        grid=(_N_TAG_BLK,),
        in_specs=[
            pl.BlockSpec((RBLK, HD), lambda g: (g, 0)),
            pl.BlockSpec((3, INTD), lambda g: (0, 0)),
            pl.BlockSpec((4, INTD, HD), lambda g: (0, 0, 0)),
            pl.BlockSpec((1, HD), lambda g: (0, 0)),
        ],
        out_specs=pl.BlockSpec((3 * RBLK, HD), lambda g: (g, 0)),
        out_shape=jax.ShapeDtypeStruct((_TI_ROWS, HD), jnp.float32),
    )(pg, emb_int, w4, b2)


@functools.lru_cache(maxsize=None)
def _make_gather_sum(ntok):
    info = plsc.get_sparse_core_info()
    nc, ns = info.num_cores, info.num_subcores
    nw = nc * ns
    tpw = ntok // nw
    C = 128
    nchunk = tpw // C
    npair = nchunk // 2
    mesh = plsc.VectorSubcoreMesh(core_axis_name="c", subcore_axis_name="s")

    @functools.partial(
        pl.kernel,
        mesh=mesh,
        compiler_params=pltpu.CompilerParams(use_tc_tiling_on_sc=False),
        out_type=jax.ShapeDtypeStruct((ntok, HD), jnp.float32),
        scratch_types=[
            pltpu.VMEM((4, C), jnp.int32),
            pltpu.VMEM((4, C), jnp.int32),
            pltpu.VMEM((C,), jnp.int32),
            pltpu.VMEM((C,), jnp.int32),
            pltpu.VMEM((C,), jnp.int32),
            pltpu.VMEM((C,), jnp.int32),
            pltpu.VMEM((C, HD), jnp.float32),
            pltpu.VMEM((C, HD), jnp.float32),
            pltpu.VMEM((C, HD), jnp.float32),
            pltpu.VMEM((C, HD), jnp.float32),
            pltpu.VMEM((C, HD), jnp.float32),
            pltpu.VMEM((C, HD), jnp.float32),
            pltpu.SemaphoreType.DMA,
            pltpu.SemaphoreType.DMA,
        ],
    )
    def gather_sum(ptq_hbm, ti_hbm, idx_hbm, out_hbm,
                   ib0, ib1, jq0, jq1, jti0, jti1,
                   rt0, rq0, rti0, rt1, rq1, rti1, gsem0, gsem1):
        wid = lax.axis_index("s") * nc + lax.axis_index("c")
        base = wid * tpw

        slots = (
            (ib0, jq0, jti0, rt0, rq0, rti0, gsem0),
            (ib1, jq1, jti1, rt1, rq1, rti1, gsem1),
        )

        def load_idx(slot, g):
            ib = slots[slot][0]
            pltpu.sync_copy(idx_hbm.at[wid, g], ib)

        def fire(slot):
            ib, jq, jti, rt, rq, rti, gsem = slots[slot]
            for j in range(C // 16):
                sl = pl.ds(j * 16, 16)
                jq[sl] = ib[1, sl] + _OFF_Q
                jti[sl] = ib[2, sl] * 3 + ib[3, sl]
            pltpu.async_copy(ptq_hbm.at[ib.at[0]], rt, gsem)
            pltpu.async_copy(ptq_hbm.at[jq], rq, gsem)
            pltpu.async_copy(ti_hbm.at[jti], rti, gsem)

        def drain(slot):
            _, _, _, rt, rq, rti, gsem = slots[slot]
            pltpu.make_async_copy(ptq_hbm.at[pl.ds(0, C)], rt, gsem).wait()
            pltpu.make_async_copy(ptq_hbm.at[pl.ds(0, C)], rq, gsem).wait()
            pltpu.make_async_copy(ptq_hbm.at[pl.ds(0, C)], rti, gsem).wait()

        def finish(slot, g):
            _, _, _, rt, rq, rti, _ = slots[slot]

            def add_one(rr, c2):
                for k in range(HD // 16):
                    sl = pl.ds(k * 16, 16)
                    rt[rr, sl] = rt[rr, sl] + rq[rr, sl] + rti[rr, sl]
                return c2

            lax.fori_loop(0, C, add_one, 0)
            pltpu.sync_copy(rt, out_hbm.at[pl.ds(base + g * C, C)])

        load_idx(0, 0)
        fire(0)

        def pair(p, carry):
            g0 = 2 * p
            g1 = g0 + 1
            load_idx(1, g1)
            fire(1)
            drain(0)
            finish(0, g0)

            @pl.when(p < npair - 1)
            def _():
                load_idx(0, g0 + 2)
                fire(0)

            drain(1)
            finish(1, g1)
            return carry

        lax.fori_loop(0, npair, pair, 0)

    return gather_sum, nw, nchunk, C


def kernel(testId, assessmentItemID, KnowledgeTag, answerCode, mask, interaction,
           emb_interaction, emb_test, emb_question, emb_tag, W, b):
    bsz, seq = interaction.shape
    ntok = bsz * seq
    gather_sum, nw, nchunk, C = _make_gather_sum(ntok)

    ii = interaction.reshape(-1).astype(jnp.int32)
    it = testId.reshape(-1).astype(jnp.int32)
    iq = assessmentItemID.reshape(-1).astype(jnp.int32)
    ig = KnowledgeTag.reshape(-1).astype(jnp.int32)
    idx4 = jnp.stack([it, iq, ig, ii]).reshape(4, nw, nchunk, C).transpose(1, 2, 0, 3)

    w4 = W.reshape(4, INTD, HD)
    ptq, pg = _project(emb_test, emb_question, emb_tag, w4)
    ti = _combine(pg, emb_interaction, w4, b.reshape(1, HD))
    Xf = gather_sum(ptq, ti, idx4)
    return (Xf.reshape(bsz, seq, HD), bsz)
